# baseline (speedup 1.0000x reference)
import functools

import jax
import jax.numpy as jnp
from jax import lax
from jax.experimental import pallas as pl
from jax.experimental.pallas import tpu as pltpu
from jax.experimental.pallas import tpu_sc as plsc

_B = 4
_N = 10000
_T = 12
_D = 64
_TIMES = 288
_DAYS = 7

_C = 80
_NCHUNK = _N // _C
_ITEMS = _NCHUNK * _B
_NW = 32

_mesh = lambda: plsc.VectorSubcoreMesh(core_axis_name="c", subcore_axis_name="s")


def _rne_bf16_bits(v):
    u = plsc.bitcast(v, jnp.uint32)
    r = u + jnp.uint32(0x7FFF) + ((u >> 16) & jnp.uint32(1))
    return r & jnp.uint32(0xFFFF0000)


_TDROWS = 2048
_TDCOLS = _D


def _idx_body(x_ref, t_ref, d_ref, o_idx, o_td):
    x = x_ref[...]
    rows = lax.broadcasted_iota(jnp.int32, (2 * _T, 3 * _T), 0)
    cols = lax.broadcasted_iota(jnp.int32, (2 * _T, 3 * _T), 1)
    sel = (rows + _T == cols).astype(jnp.float32)
    xt = lax.dot_general(sel, x, (((1,), (1,)), ((), ())),
                         precision=lax.Precision.HIGHEST,
                         preferred_element_type=jnp.float32)
    v1 = xt[:_T, :]
    v2 = xt[_T:, :]
    o_idx[...] = (v1 * float(_TIMES)).astype(jnp.int32) * _DAYS \
        + v2.astype(jnp.int32)
    ri = lax.broadcasted_iota(jnp.int32, (_TDROWS, _TIMES), 0)
    ci = lax.broadcasted_iota(jnp.int32, (_TDROWS, _TIMES), 1)
    selt = (ri // _DAYS == ci).astype(jnp.float32)
    rj = lax.broadcasted_iota(jnp.int32, (_TDROWS, _DAYS), 0)
    cj = lax.broadcasted_iota(jnp.int32, (_TDROWS, _DAYS), 1)
    seld = (rj - _DAYS * (rj // _DAYS) == cj).astype(jnp.float32)
    tdt = lax.dot_general(selt, t_ref[...], (((1,), (0,)), ((), ())),
                          precision=lax.Precision.HIGHEST,
                          preferred_element_type=jnp.float32)
    tdd = lax.dot_general(seld, d_ref[...], (((1,), (0,)), ((), ())),
                          precision=lax.Precision.HIGHEST,
                          preferred_element_type=jnp.float32)
    o_td[...] = tdt * tdd


_idx_build = pl.pallas_call(
    _idx_body,
    out_shape=[
        jax.ShapeDtypeStruct((_T, _B * _N), jnp.int32),
        jax.ShapeDtypeStruct((_TDROWS, _TDCOLS), jnp.float32),
    ],
)


@functools.partial(
    pl.kernel,
    out_type=jax.ShapeDtypeStruct((_B, _T, _N, _D), jnp.float32),
    mesh=_mesh(),
    compiler_params=pltpu.CompilerParams(use_tc_tiling_on_sc=False),
    scratch_types=[
        pltpu.VMEM((_C,), jnp.int32),
        pltpu.VMEM((_C,), jnp.int32),
        pltpu.VMEM((_C,), jnp.int32),
        pltpu.VMEM((_C, _TDCOLS), jnp.float32),
        pltpu.VMEM((_C, _TDCOLS), jnp.float32),
        pltpu.VMEM((_C, _D), jnp.float32),
        pltpu.VMEM((_C, _D), jnp.float32),
        pltpu.VMEM((_C, _D), jnp.float32),
        pltpu.SemaphoreType.DMA,
        pltpu.SemaphoreType.DMA,
        pltpu.SemaphoreType.DMA,
        pltpu.SemaphoreType.DMA,
    ],
)
def _emb_lookup(cidx, node, td, out, tixa, tixb, tixc, tra, trb, ora, orb,
                nrows, gsem, nsem, wsem, isem):
    wid = lax.axis_index("s") * 2 + lax.axis_index("c")
    nitems = (_ITEMS + _NW - 1 - wid) // _NW
    tix = (tixa, tixb, tixc)
    tr = (tra, trb)
    orw = (ora, orb)

    def item_body(k, carry):
        i = wid + _NW * k
        chunk = i // _B
        b = i - _B * chunk
        n0 = chunk * _C
        col0 = b * _N + n0
        ncopy = pltpu.async_copy(node.at[pl.ds(n0, _C)], nrows, nsem)
        ics = [None] * _T
        gs = [None] * _T
        ws = [None] * _T
        for t in (0, 1):
            ics[t] = pltpu.async_copy(
                cidx.at[pl.ds(t * _B * _N + col0, _C)], tix[t], isem)
        ics[0].wait()
        gs[0] = pltpu.async_copy(td.at[tix[0]], tr[0], gsem)
        ncopy.wait()
        for t in range(_T):
            cur = t % 2
            if t + 2 < _T:
                ics[t + 2] = pltpu.async_copy(
                    cidx.at[pl.ds((t + 2) * _B * _N + col0, _C)],
                    tix[(t + 2) % 3], isem)
            if t + 1 < _T:
                ics[t + 1].wait()
                gs[t + 1] = pltpu.async_copy(td.at[tix[(t + 1) % 3]],
                                             tr[1 - cur], gsem)
            gs[t].wait()
            if t >= 2:
                ws[t - 2].wait()

            def r_body(rb, c3):
                for u in range(8):
                    r = rb * 8 + u
                    for c in range(_D // 16):
                        sl = pl.ds(16 * c, 16)
                        orw[cur][r, sl] = tr[cur][r, sl] * nrows[r, sl]
                return c3

            lax.fori_loop(0, _C // 8, r_body, 0)
            ws[t] = pltpu.async_copy(orw[cur],
                                     out.at[b, t, pl.ds(n0, _C)],
                                     wsem)
        ws[_T - 2].wait()
        ws[_T - 1].wait()
        return carry

    lax.fori_loop(0, nitems, item_body, 0)


def kernel(x, node_embeddings1, T_i_D_emb, D_i_W_emb):
    cidx, td = _idx_build(x.reshape(_B * _N, 3 * _T), T_i_D_emb, D_i_W_emb)
    out = _emb_lookup(cidx.reshape(-1), node_embeddings1, td)
    return out, node_embeddings1


# speedup vs baseline: 1.6124x; 1.6124x over previous
import functools

import jax
import jax.numpy as jnp
from jax import lax
from jax.experimental import pallas as pl
from jax.experimental.pallas import tpu as pltpu
from jax.experimental.pallas import tpu_sc as plsc

_B = 4
_N = 10000
_T = 12
_D = 64
_TIMES = 288
_DAYS = 7

_C = 80
_NCHUNK = _N // _C
_ITEMS = _NCHUNK * _B
_NW = 32

_mesh = lambda: plsc.VectorSubcoreMesh(core_axis_name="c", subcore_axis_name="s")


def _rne_bf16_bits(v):
    u = plsc.bitcast(v, jnp.uint32)
    r = u + jnp.uint32(0x7FFF) + ((u >> 16) & jnp.uint32(1))
    return r & jnp.uint32(0xFFFF0000)


_TDROWS = 2048
_TDCOLS = _D


def _idx_body(x_ref, t_ref, d_ref, o_idx, o_td):
    x = x_ref[...]
    rows = lax.broadcasted_iota(jnp.int32, (2 * _T, 3 * _T), 0)
    cols = lax.broadcasted_iota(jnp.int32, (2 * _T, 3 * _T), 1)
    sel = (rows + _T == cols).astype(jnp.float32)
    xt = lax.dot_general(sel, x, (((1,), (1,)), ((), ())),
                         precision=lax.Precision.HIGHEST,
                         preferred_element_type=jnp.float32)
    v1 = xt[:_T, :]
    v2 = xt[_T:, :]
    o_idx[...] = (v1 * float(_TIMES)).astype(jnp.int32) * _DAYS \
        + v2.astype(jnp.int32)
    ri = lax.broadcasted_iota(jnp.int32, (_TDROWS, _TIMES), 0)
    ci = lax.broadcasted_iota(jnp.int32, (_TDROWS, _TIMES), 1)
    selt = (ri // _DAYS == ci).astype(jnp.float32)
    rj = lax.broadcasted_iota(jnp.int32, (_TDROWS, _DAYS), 0)
    cj = lax.broadcasted_iota(jnp.int32, (_TDROWS, _DAYS), 1)
    seld = (rj - _DAYS * (rj // _DAYS) == cj).astype(jnp.float32)
    tdt = lax.dot_general(selt, t_ref[...], (((1,), (0,)), ((), ())),
                          precision=lax.Precision.HIGHEST,
                          preferred_element_type=jnp.float32)
    tdd = lax.dot_general(seld, d_ref[...], (((1,), (0,)), ((), ())),
                          precision=lax.Precision.HIGHEST,
                          preferred_element_type=jnp.float32)
    o_td[...] = tdt * tdd


_idx_build = pl.pallas_call(
    _idx_body,
    out_shape=[
        jax.ShapeDtypeStruct((_T, _B * _N), jnp.int32),
        jax.ShapeDtypeStruct((_TDROWS, _TDCOLS), jnp.float32),
    ],
)


@functools.partial(
    pl.kernel,
    out_type=jax.ShapeDtypeStruct((_B, _T, _N, _D), jnp.float32),
    mesh=_mesh(),
    scratch_types=[
        pltpu.VMEM((_C,), jnp.int32),
        pltpu.VMEM((_C,), jnp.int32),
        pltpu.VMEM((_C,), jnp.int32),
        pltpu.VMEM((_C, _TDCOLS), jnp.float32),
        pltpu.VMEM((_C, _TDCOLS), jnp.float32),
        pltpu.VMEM((_C, _D), jnp.float32),
        pltpu.VMEM((_C, _D), jnp.float32),
        pltpu.VMEM((_C, _D), jnp.float32),
        pltpu.VMEM_SHARED((_TDROWS, _TDCOLS), jnp.float32),
        pltpu.SemaphoreType.DMA,
        pltpu.SemaphoreType.DMA,
        pltpu.SemaphoreType.DMA,
        pltpu.SemaphoreType.DMA,
    ],
)
def _emb_lookup(cidx, node, td, out, tixa, tixb, tixc, tra, trb, ora, orb,
                nrows, tdsh, gsem, nsem, wsem, isem):
    sid = lax.axis_index("s")
    wid = sid * 2 + lax.axis_index("c")
    nitems = (_ITEMS + _NW - 1 - wid) // _NW
    tix = (tixa, tixb, tixc)
    tr = (tra, trb)
    orw = (ora, orb)

    pltpu.sync_copy(td.at[pl.ds(sid * (_TDROWS // 16), _TDROWS // 16)],
                    tdsh.at[pl.ds(sid * (_TDROWS // 16), _TDROWS // 16)])
    plsc.subcore_barrier()

    def item_body(k, carry):
        i = wid + _NW * k
        chunk = i // _B
        b = i - _B * chunk
        n0 = chunk * _C
        col0 = b * _N + n0
        ncopy = pltpu.async_copy(node.at[pl.ds(n0, _C)], nrows, nsem)
        ics = [None] * _T
        gs = [None] * _T
        ws = [None] * _T
        for t in (0, 1):
            ics[t] = pltpu.async_copy(
                cidx.at[pl.ds(t * _B * _N + col0, _C)], tix[t], isem)
        ics[0].wait()
        gs[0] = pltpu.async_copy(tdsh.at[tix[0]], tr[0], gsem)
        ncopy.wait()
        for t in range(_T):
            cur = t % 2
            if t + 2 < _T:
                ics[t + 2] = pltpu.async_copy(
                    cidx.at[pl.ds((t + 2) * _B * _N + col0, _C)],
                    tix[(t + 2) % 3], isem)
            if t + 1 < _T:
                ics[t + 1].wait()
                gs[t + 1] = pltpu.async_copy(tdsh.at[tix[(t + 1) % 3]],
                                             tr[1 - cur], gsem)
            gs[t].wait()
            if t >= 2:
                ws[t - 2].wait()

            def r_body(rb, c3):
                for u in range(8):
                    r = rb * 8 + u
                    for c in range(_D // 16):
                        sl = pl.ds(16 * c, 16)
                        orw[cur][r, sl] = tr[cur][r, sl] * nrows[r, sl]
                return c3

            lax.fori_loop(0, _C // 8, r_body, 0)
            ws[t] = pltpu.async_copy(orw[cur],
                                     out.at[b, t, pl.ds(n0, _C)],
                                     wsem)
        ws[_T - 2].wait()
        ws[_T - 1].wait()
        return carry

    lax.fori_loop(0, nitems, item_body, 0)


def kernel(x, node_embeddings1, T_i_D_emb, D_i_W_emb):
    cidx, td = _idx_build(x.reshape(_B * _N, 3 * _T), T_i_D_emb, D_i_W_emb)
    out = _emb_lookup(cidx.reshape(-1), node_embeddings1, td)
    return out, node_embeddings1


# speedup vs baseline: 1.6136x; 1.0008x over previous
import functools

import jax
import jax.numpy as jnp
from jax import lax
from jax.experimental import pallas as pl
from jax.experimental.pallas import tpu as pltpu
from jax.experimental.pallas import tpu_sc as plsc

_B = 4
_N = 10000
_T = 12
_D = 64
_TIMES = 288
_DAYS = 7

_C = 80
_NCHUNK = _N // _C
_ITEMS = _NCHUNK * _B
_NW = 32

_mesh = lambda: plsc.VectorSubcoreMesh(core_axis_name="c", subcore_axis_name="s")


_TDROWS = 2048
_TDCOLS = _D


def _idx_body(x_ref, t_ref, d_ref, o_idx, o_td):
    x = x_ref[...]
    rows = lax.broadcasted_iota(jnp.int32, (2 * _T, 3 * _T), 0)
    cols = lax.broadcasted_iota(jnp.int32, (2 * _T, 3 * _T), 1)
    sel = (rows + _T == cols).astype(jnp.float32)
    xt = lax.dot_general(sel, x, (((1,), (1,)), ((), ())),
                         precision=lax.Precision.HIGHEST,
                         preferred_element_type=jnp.float32)
    v1 = xt[:_T, :]
    v2 = xt[_T:, :]
    o_idx[...] = (v1 * float(_TIMES)).astype(jnp.int32) * _DAYS \
        + v2.astype(jnp.int32)
    ri = lax.broadcasted_iota(jnp.int32, (_TDROWS, _TIMES), 0)
    ci = lax.broadcasted_iota(jnp.int32, (_TDROWS, _TIMES), 1)
    selt = (ri // _DAYS == ci).astype(jnp.float32)
    rj = lax.broadcasted_iota(jnp.int32, (_TDROWS, _DAYS), 0)
    cj = lax.broadcasted_iota(jnp.int32, (_TDROWS, _DAYS), 1)
    seld = (rj - _DAYS * (rj // _DAYS) == cj).astype(jnp.float32)
    tdt = lax.dot_general(selt, t_ref[...], (((1,), (0,)), ((), ())),
                          precision=lax.Precision.HIGHEST,
                          preferred_element_type=jnp.float32)
    tdd = lax.dot_general(seld, d_ref[...], (((1,), (0,)), ((), ())),
                          precision=lax.Precision.HIGHEST,
                          preferred_element_type=jnp.float32)
    o_td[...] = tdt * tdd


_idx_build = pl.pallas_call(
    _idx_body,
    out_shape=[
        jax.ShapeDtypeStruct((_T, _B * _N), jnp.int32),
        jax.ShapeDtypeStruct((_TDROWS, _TDCOLS), jnp.float32),
    ],
)


@functools.partial(
    pl.kernel,
    out_type=jax.ShapeDtypeStruct((_B, _T, _N, _D), jnp.float32),
    mesh=_mesh(),
    scratch_types=[
        pltpu.VMEM((_C,), jnp.int32),
        pltpu.VMEM((_C,), jnp.int32),
        pltpu.VMEM((_C,), jnp.int32),
        pltpu.VMEM((_C, _TDCOLS), jnp.float32),
        pltpu.VMEM((_C, _TDCOLS), jnp.float32),
        pltpu.VMEM((_C, _D), jnp.float32),
        pltpu.VMEM((_C, _D), jnp.float32),
        pltpu.VMEM((_C, _D), jnp.float32),
        pltpu.VMEM_SHARED((_TDROWS, _TDCOLS), jnp.float32),
        pltpu.SemaphoreType.DMA,
        pltpu.SemaphoreType.DMA,
        pltpu.SemaphoreType.DMA,
        pltpu.SemaphoreType.DMA,
    ],
)
def _emb_lookup(cidx, node, td, out, tixa, tixb, tixc, tra, trb, ora, orb,
                nrows, tdsh, gsem, nsem, wsem, isem):
    sid = lax.axis_index("s")
    wid = sid * 2 + lax.axis_index("c")
    nitems = (_ITEMS + _NW - 1 - wid) // _NW
    tix = (tixa, tixb, tixc)
    tr = (tra, trb)
    orw = (ora, orb)

    pltpu.sync_copy(td.at[pl.ds(sid * (_TDROWS // 16), _TDROWS // 16)],
                    tdsh.at[pl.ds(sid * (_TDROWS // 16), _TDROWS // 16)])
    plsc.subcore_barrier()

    def item_body(k, carry):
        i = wid + _NW * k
        chunk = i // _B
        b = i - _B * chunk
        n0 = chunk * _C
        col0 = b * _N + n0
        ncopy = pltpu.async_copy(node.at[pl.ds(n0, _C)], nrows, nsem)
        ics = [None] * _T
        gs = [None] * _T
        ws = [None] * _T
        for t in (0, 1):
            ics[t] = pltpu.async_copy(
                cidx.at[pl.ds(t * _B * _N + col0, _C)], tix[t], isem)
        ics[0].wait()
        gs[0] = pltpu.async_copy(tdsh.at[tix[0]], tr[0], gsem)
        ncopy.wait()
        for t in range(_T):
            cur = t % 2
            if t + 2 < _T:
                ics[t + 2] = pltpu.async_copy(
                    cidx.at[pl.ds((t + 2) * _B * _N + col0, _C)],
                    tix[(t + 2) % 3], isem)
            if t + 1 < _T:
                ics[t + 1].wait()
                gs[t + 1] = pltpu.async_copy(tdsh.at[tix[(t + 1) % 3]],
                                             tr[1 - cur], gsem)
            gs[t].wait()
            if t >= 2:
                ws[t - 2].wait()

            def r_body(rb, c3):
                for u in range(8):
                    r = rb * 8 + u
                    for c in range(_D // 16):
                        sl = pl.ds(16 * c, 16)
                        orw[cur][r, sl] = tr[cur][r, sl] * nrows[r, sl]
                return c3

            lax.fori_loop(0, _C // 8, r_body, 0)
            ws[t] = pltpu.async_copy(orw[cur],
                                     out.at[b, t, pl.ds(n0, _C)],
                                     wsem)
        ws[_T - 2].wait()
        ws[_T - 1].wait()
        return carry

    lax.fori_loop(0, nitems, item_body, 0)


def kernel(x, node_embeddings1, T_i_D_emb, D_i_W_emb):
    cidx, td = _idx_build(x.reshape(_B * _N, 3 * _T), T_i_D_emb, D_i_W_emb)
    out = _emb_lookup(cidx.reshape(-1), node_embeddings1, td)
    return out, node_embeddings1
